# baseline (speedup 1.0000x reference)
import functools

import jax
import jax.numpy as jnp
from jax import lax
from jax.experimental import pallas as pl
from jax.experimental.pallas import tpu as pltpu
from jax.experimental.pallas import tpu_sc as plsc

L = 16
N_WORKERS = 32
C = 64
NSLOTS = 4


def _make_kernel(B, K):
    bpw = B // N_WORKERS
    nchunks = bpw // C
    mesh = plsc.VectorSubcoreMesh(core_axis_name="c", subcore_axis_name="s")

    @functools.partial(
        pl.kernel,
        out_type=jax.ShapeDtypeStruct((B,), jnp.float32),
        mesh=mesh,
        compiler_params=pltpu.CompilerParams(needs_layout_passes=False),
        scratch_types=[
            pltpu.VMEM((bpw,), jnp.int32),
            pltpu.VMEM((bpw,), jnp.int32),
            pltpu.VMEM((NSLOTS, C, K), jnp.float32),
            pltpu.VMEM((NSLOTS, C, K), jnp.float32),
            pltpu.VMEM((bpw,), jnp.float32),
        ] + [pltpu.SemaphoreType.DMA] * (2 * NSLOTS),
    )
    def deep_mf(pu_hbm, qi_hbm, users_hbm, items_hbm, out_hbm,
                uidx_v, iidx_v, ubuf_v, ibuf_v, out_v, *sems):
        wid = lax.axis_index("s") * 2 + lax.axis_index("c")
        usems = sems[:NSLOTS]
        isems = sems[NSLOTS:]

        pltpu.sync_copy(users_hbm.at[wid], uidx_v)
        pltpu.sync_copy(items_hbm.at[wid], iidx_v)

        def fire(chunk, slot):
            ub = ubuf_v.at[slot]
            ib = ibuf_v.at[slot]
            usem = usems[slot]
            isem = isems[slot]

            def body(g, carry):
                base = chunk * C + g * L
                uvec = uidx_v[pl.ds(base, L)]
                ivec = iidx_v[pl.ds(base, L)]
                for k in range(L):
                    row = g * L + k
                    pltpu.async_copy(
                        pu_hbm.at[pl.ds(uvec[k], 1)],
                        ub.at[pl.ds(row, 1)], usem)
                    pltpu.async_copy(
                        qi_hbm.at[pl.ds(ivec[k], 1)],
                        ib.at[pl.ds(row, 1)], isem)
                return carry

            lax.fori_loop(0, C // L, body, 0)

        def drain(slot):
            pltpu.make_async_copy(
                pu_hbm.at[pl.ds(0, C)], ubuf_v.at[slot], usems[slot]).wait()
            pltpu.make_async_copy(
                qi_hbm.at[pl.ds(0, C)], ibuf_v.at[slot], isems[slot]).wait()

        lane = lax.iota(jnp.int32, L)

        def reduce_chunk(chunk, slot):
            ub = ubuf_v.at[slot]
            ib = ibuf_v.at[slot]

            def body(g, carry):
                rows = g * L + lane
                acc = jnp.zeros((L,), jnp.float32)
                for j in range(K):
                    col = jnp.full((L,), j, jnp.int32)
                    uj = plsc.load_gather(ub, [rows, col])
                    vj = plsc.load_gather(ib, [rows, col])
                    acc = acc + uj * vj
                out_v[pl.ds(chunk * C + g * L, L)] = acc
                return carry

            lax.fori_loop(0, C // L, body, 0)

        for chunk in range(min(NSLOTS, nchunks)):
            fire(chunk, chunk)
        for chunk in range(nchunks):
            slot = chunk % NSLOTS
            drain(slot)
            reduce_chunk(chunk, slot)
            if chunk + NSLOTS < nchunks:
                fire(chunk + NSLOTS, slot)

        pltpu.sync_copy(out_v, out_hbm.at[pl.ds(wid * bpw, bpw)])

    return deep_mf


@jax.jit
def kernel(users, items, pu_table, qi_table):
    B = users.shape[0]
    K = pu_table.shape[1]
    users2d = users.reshape(-1).astype(jnp.int32).reshape(N_WORKERS, -1)
    items2d = items.reshape(-1).astype(jnp.int32).reshape(N_WORKERS, -1)
    out = _make_kernel(B, K)(pu_table, qi_table, users2d, items2d)
    return out.reshape(B, 1)
